# baseline (speedup 1.0000x reference)
import functools

import jax
import jax.numpy as jnp
from jax import lax
from jax.experimental import pallas as pl
from jax.experimental.pallas import tpu as pltpu
from jax.experimental.pallas import tpu_sc as plsc

_LANES = 16


def _make_sc_kernel(B, S, D, n_workers):
    rows = 8
    dh = D // 2
    assert S % (n_workers * rows) == 0
    n_sblk = S // (n_workers * rows)
    n_combo = n_sblk * 2

    mesh = plsc.VectorSubcoreMesh(core_axis_name="c", subcore_axis_name="s")
    num_cores = mesh.num_cores

    @functools.partial(
        pl.kernel,
        out_type=jax.ShapeDtypeStruct((B, S, D), jnp.float32),
        mesh=mesh,
        scratch_types=[
            [pltpu.VMEM((rows, dh), jnp.float32) for _ in range(2)],
            [pltpu.VMEM((rows, dh), jnp.float32) for _ in range(B)],
            [pltpu.SemaphoreType.DMA for _ in range(2)],
            [pltpu.SemaphoreType.DMA for _ in range(B)],
            [pltpu.SemaphoreType.DMA for _ in range(B)],
        ],
        compiler_params=pltpu.CompilerParams(skip_device_barrier=True),
    )
    def sc_add(x_hbm, pos_hbm, out_hbm, pos_v, x_v, psem, xsem, osem):
        wid = lax.axis_index("s") * num_cores + lax.axis_index("c")
        base_s = wid * n_sblk * rows

        def s0(c):
            return base_s + c * rows

        def pos_cp(c, h, slot):
            return pltpu.make_async_copy(
                pos_hbm.at[pl.ds(s0(c), rows), pl.ds(h * dh, dh)],
                pos_v[slot],
                psem[slot],
            )

        def x_cp(c, h, b):
            return pltpu.make_async_copy(
                x_hbm.at[b, pl.ds(s0(c), rows), pl.ds(h * dh, dh)],
                x_v[b],
                xsem[b],
            )

        def o_cp(c, h, b):
            return pltpu.make_async_copy(
                x_v[b],
                out_hbm.at[b, pl.ds(s0(c), rows), pl.ds(h * dh, dh)],
                osem[b],
            )

        pos_cp(0, 0, 0).start()
        x_cp(0, 0, 0).start()
        x_cp(0, 0, 1).start()

        @pl.loop(0, n_sblk)
        def sblk_loop(c):
            for h in range(2):
                for b in range(B):
                    if b == 0:
                        pos_cp(c, h, h).wait()
                        if h == 0:
                            pos_cp(c, 1, 1).start()
                        else:

                            @pl.when(c + 1 < n_sblk)
                            def _():
                                pos_cp(c + 1, 0, 0).start()

                    x_cp(c, h, b).wait()

                    s2 = (b + 2) % B
                    if b < 2:
                        if h == 1:
                            o_cp(c, 0, s2).wait()
                        else:

                            @pl.when(c >= 1)
                            def _():
                                o_cp(c - 1, 1, s2).wait()

                        x_cp(c, h, s2).start()
                    else:
                        o_cp(c, h, s2).wait()
                        if h == 0:
                            x_cp(c, 1, s2).start()
                        else:

                            @pl.when(c + 1 < n_sblk)
                            def _():
                                x_cp(c + 1, 0, s2).start()

                    for r in range(rows):

                        @plsc.parallel_loop(0, dh, step=_LANES, unroll=8)
                        def add_body(i):
                            v = pl.ds(i, _LANES)
                            x_v[b][r, v] = x_v[b][r, v] + pos_v[h][r, v]

                    o_cp(c, h, b).start()

        for b in range(B - 2, B):
            o_cp(n_sblk - 1, 1, b).wait()

    return sc_add


def kernel(x, pos_table):
    B, S, D = x.shape
    sc_add = _make_sc_kernel(B, S, D, n_workers=32)
    return sc_add(x, pos_table)
